# baseline (speedup 1.0000x reference)
import jax
import jax.numpy as jnp
from jax import lax
from jax.experimental import pallas as pl
from jax.experimental.pallas import tpu as pltpu
from jax.experimental.pallas import tpu_sc as plsc

_NC = 2
_NS = 16
_NW = _NC * _NS
_CHUNK = 16
_NBUF = 7
_LEAD = 4

_TC_ROWS = 4096
_TC_SEMS = 16


def _emb_body(idx_hbm, table_hbm, out_hbm, idx_v, *bufs_and_sems):
    rows = bufs_and_sems[:_NBUF]
    gsems = bufs_and_sems[_NBUF:2 * _NBUF]
    wsems = bufs_and_sems[2 * _NBUF:3 * _NBUF]

    wid = lax.axis_index("s") * _NC + lax.axis_index("c")
    n_chunks = idx_hbm.shape[1]
    pltpu.sync_copy(idx_hbm.at[wid], idx_v)
    row_base = wid * n_chunks * _CHUNK

    def gather(i):
        k = i % _NBUF
        pltpu.async_copy(table_hbm.at[idx_v.at[i]], rows[k], gsems[k])

    def wait_gather(i):
        k = i % _NBUF
        pltpu.make_async_copy(table_hbm.at[idx_v.at[i]], rows[k], gsems[k]).wait()

    def write(i):
        k = i % _NBUF
        pltpu.async_copy(
            rows[k], out_hbm.at[pl.ds(row_base + i * _CHUNK, _CHUNK)], wsems[k])

    def wait_write(i):
        k = i % _NBUF
        pltpu.make_async_copy(
            rows[k], out_hbm.at[pl.ds(row_base + i * _CHUNK, _CHUNK)], wsems[k]).wait()

    for c in range(_LEAD):
        gather(c)
    for c in range(n_chunks):
        wait_gather(c)
        write(c)
        j = c + _LEAD
        if j < n_chunks:
            if j >= _NBUF:
                wait_write(j - _NBUF)
            gather(j)
    for c in range(n_chunks - _NBUF, n_chunks):
        wait_write(c)


def _tc_body(idx_ref, table_hbm, out_hbm, sems):
    n = idx_ref.shape[0]

    def issue(r, carry):
        pltpu.async_copy(
            table_hbm.at[idx_ref[r]], out_hbm.at[r], sems.at[r % _TC_SEMS])
        return carry

    def steady(r, carry):
        pltpu.make_async_copy(
            table_hbm.at[0], out_hbm.at[r - _TC_SEMS],
            sems.at[r % _TC_SEMS]).wait()
        pltpu.async_copy(
            table_hbm.at[idx_ref[r]], out_hbm.at[r], sems.at[r % _TC_SEMS])
        return carry

    def drain(r, carry):
        pltpu.make_async_copy(
            table_hbm.at[0], out_hbm.at[r], sems.at[r % _TC_SEMS]).wait()
        return carry

    lax.fori_loop(0, _TC_SEMS, issue, 0)
    lax.fori_loop(_TC_SEMS, n, steady, 0)
    lax.fori_loop(n - _TC_SEMS, n, drain, 0)


def _sc_gather(idx, table, n_tok, d):
    n_chunks = n_tok // (_NW * _CHUNK)
    idx3 = idx.reshape(_NW, n_chunks, _CHUNK)
    mesh = plsc.VectorSubcoreMesh(core_axis_name="c", subcore_axis_name="s")
    fn = pl.kernel(
        _emb_body,
        out_type=jax.ShapeDtypeStruct((n_tok, d), jnp.float32),
        mesh=mesh,
        scratch_types=(
            [pltpu.VMEM((n_chunks, _CHUNK), jnp.int32)]
            + [pltpu.VMEM((_CHUNK, d), jnp.float32) for _ in range(_NBUF)]
            + [pltpu.SemaphoreType.DMA for _ in range(2 * _NBUF)]
        ),
    )
    return fn(idx3, table)


def _tc_gather(idx, table, d):
    n = idx.shape[0]
    return pl.pallas_call(
        _tc_body,
        out_shape=jax.ShapeDtypeStruct((n, d), jnp.float32),
        in_specs=[
            pl.BlockSpec(memory_space=pltpu.SMEM),
            pl.BlockSpec(memory_space=pl.ANY),
        ],
        out_specs=pl.BlockSpec(memory_space=pl.ANY),
        scratch_shapes=[pltpu.SemaphoreType.DMA((_TC_SEMS,))],
    )(idx, table)


def kernel(input_ids, embedding_weight):
    b, s = input_ids.shape
    _, d = embedding_weight.shape
    n_tok = b * s
    n_sc = n_tok - _TC_ROWS
    idx = input_ids.astype(jnp.int32).reshape(n_tok)

    out_sc = _sc_gather(idx[:n_sc], embedding_weight, n_sc, d)
    out_tc = _tc_gather(idx[n_sc:], embedding_weight, d)
    out = jnp.concatenate([out_sc, out_tc], axis=0)
    return out.reshape(b, s, d)


# speedup vs baseline: 8.6805x; 8.6805x over previous
import jax
import jax.numpy as jnp
from jax import lax
from jax.experimental import pallas as pl
from jax.experimental.pallas import tpu as pltpu
from jax.experimental.pallas import tpu_sc as plsc

_NC = 2
_NS = 16
_NW = _NC * _NS
_CHUNK = 16
_NBUF = 7
_LEAD = 5


def _emb_body(idx_hbm, table_hbm, out_hbm, idx_v, *bufs_and_sems):
    rows = bufs_and_sems[:_NBUF]
    gsems = bufs_and_sems[_NBUF:2 * _NBUF]
    wsems = bufs_and_sems[2 * _NBUF:3 * _NBUF]

    wid = lax.axis_index("s") * _NC + lax.axis_index("c")
    n_chunks = idx_hbm.shape[1]
    pltpu.sync_copy(idx_hbm.at[wid], idx_v)
    row_base = wid * n_chunks * _CHUNK

    def gather(i):
        k = i % _NBUF
        pltpu.async_copy(table_hbm.at[idx_v.at[i]], rows[k], gsems[k])

    def wait_gather(i):
        k = i % _NBUF
        pltpu.make_async_copy(table_hbm.at[idx_v.at[i]], rows[k], gsems[k]).wait()

    def write(i):
        k = i % _NBUF
        pltpu.async_copy(
            rows[k], out_hbm.at[pl.ds(row_base + i * _CHUNK, _CHUNK)], wsems[k])

    def wait_write(i):
        k = i % _NBUF
        pltpu.make_async_copy(
            rows[k], out_hbm.at[pl.ds(row_base + i * _CHUNK, _CHUNK)], wsems[k]).wait()

    for c in range(_LEAD):
        gather(c)
    for c in range(n_chunks):
        wait_gather(c)
        write(c)
        j = c + _LEAD
        if j < n_chunks:
            if j >= _NBUF:
                wait_write(j - _NBUF)
            gather(j)
    for c in range(n_chunks - _NBUF, n_chunks):
        wait_write(c)


def kernel(input_ids, embedding_weight):
    b, s = input_ids.shape
    _, d = embedding_weight.shape
    n_tok = b * s
    n_chunks = n_tok // (_NW * _CHUNK)
    idx = input_ids.astype(jnp.int32).reshape(_NW, n_chunks, _CHUNK)

    mesh = plsc.VectorSubcoreMesh(core_axis_name="c", subcore_axis_name="s")
    fn = pl.kernel(
        _emb_body,
        out_type=jax.ShapeDtypeStruct((n_tok, d), jnp.float32),
        mesh=mesh,
        scratch_types=(
            [pltpu.VMEM((n_chunks, _CHUNK), jnp.int32)]
            + [pltpu.VMEM((_CHUNK, d), jnp.float32) for _ in range(_NBUF)]
            + [pltpu.SemaphoreType.DMA for _ in range(2 * _NBUF)]
        ),
    )
    out = fn(idx, embedding_weight)
    return out.reshape(b, s, d)
